# Initial kernel scaffold; baseline (speedup 1.0000x reference)
import jax
import jax.numpy as jnp
from jax.experimental import pallas as pl


def kernel(x, edge_index, W1_root, W1_rel, b1, g1, bt1, W2_root, W2_rel, b2, g2, bt2, Wl1, bl1, Wl2, bl2):
    raise NotImplementedError("write your pallas kernel here")


# speedup vs baseline: 10.0952x; 10.0952x over previous
import functools

import jax
import jax.numpy as jnp
from jax import lax
from jax.experimental import pallas as pl
from jax.experimental.pallas import tpu as pltpu
from jax.experimental.pallas import tpu_sc as plsc

N_BUS = 1000
BATCH = 10
N = N_BUS * BATCH
E = 320000
FP = 16
EPS = 1e-5

NC = 2
NS = 16
NW = NC * NS
EPW = E // NW
CH = 80
NSTEP = EPW // CH
NPAD = 10240
RPW = NPAD // NS


@functools.cache
def _make_segsum_sc():
    mesh = plsc.VectorSubcoreMesh(core_axis_name="c", subcore_axis_name="s")
    return functools.partial(
        pl.kernel,
        mesh=mesh,
        compiler_params=pltpu.CompilerParams(use_tc_tiling_on_sc=False),
        out_type=jax.ShapeDtypeStruct((NC, NPAD, FP), jnp.float32),
        scratch_types=[
            pltpu.VMEM((NSTEP, CH), jnp.int32),
            pltpu.VMEM((NSTEP, CH), jnp.int32),
            pltpu.VMEM((CH, FP), jnp.float32),
            pltpu.VMEM_SHARED((NPAD, FP), jnp.float32),
            pltpu.SemaphoreType.DMA,
        ],
    )(_segsum_body)


def _segsum_body(m_hbm, src_hbm, dst_hbm, zeros_hbm, out_hbm,
                 src_v, dst_v, rows_v, acc_sh, sem):
    c = lax.axis_index("c")
    s = lax.axis_index("s")
    wid = c * NS + s

    pltpu.sync_copy(src_hbm.at[wid], src_v)
    pltpu.sync_copy(dst_hbm.at[wid], dst_v)

    pltpu.sync_copy(zeros_hbm.at[pl.ds(s * RPW, RPW)],
                    acc_sh.at[pl.ds(s * RPW, RPW)])
    plsc.subcore_barrier()

    def step(j, carry):
        pltpu.async_copy(m_hbm.at[src_v.at[j]], rows_v, sem).wait()
        pltpu.sync_copy(rows_v, acc_sh.at[dst_v.at[j]], add=True)
        return carry

    lax.fori_loop(0, NSTEP, step, 0)
    plsc.subcore_barrier()

    pltpu.sync_copy(acc_sh.at[pl.ds(s * RPW, RPW)],
                    out_hbm.at[c, pl.ds(s * RPW, RPW)])


def _proj_body(x_ref, wrel_ref, wroot_ref, m_ref, r_ref):
    x = x_ref[...]
    m_ref[...] = jnp.dot(x, wrel_ref[...], preferred_element_type=jnp.float32, precision=lax.Precision.HIGHEST)
    r_ref[...] = jnp.dot(x, wroot_ref[...], preferred_element_type=jnp.float32, precision=lax.Precision.HIGHEST)


def _bn_relu(h, g, bt):
    mu = jnp.mean(h, axis=0, keepdims=True)
    var = jnp.mean(h * h, axis=0, keepdims=True) - mu * mu
    hn = g * (h - mu) * lax.rsqrt(var + EPS) + bt
    return jnp.maximum(hn, 0.0)


def _mid_body(r_ref, p0_ref, p1_ref, b_ref, g_ref, bt_ref,
              wrel_ref, wroot_ref, m2_ref, r2_ref):
    h = r_ref[...] + p0_ref[...] + p1_ref[...] + b_ref[...]
    h1 = _bn_relu(h, g_ref[...], bt_ref[...])
    m2_ref[...] = jnp.dot(h1, wrel_ref[...], preferred_element_type=jnp.float32, precision=lax.Precision.HIGHEST)
    r2_ref[...] = jnp.dot(h1, wroot_ref[...], preferred_element_type=jnp.float32, precision=lax.Precision.HIGHEST)


def _last_body(r_ref, p0_ref, p1_ref, b_ref, g_ref, bt_ref, h2_ref):
    h = r_ref[...] + p0_ref[...] + p1_ref[...] + b_ref[...]
    h2_ref[...] = _bn_relu(h, g_ref[...], bt_ref[...])


def _head_body(hf_ref, wl1_ref, bl1_ref, wl2_ref, bl2_ref, out_ref):
    hid = jnp.dot(hf_ref[...], wl1_ref[...], preferred_element_type=jnp.float32, precision=lax.Precision.HIGHEST)
    hid = jnp.maximum(hid + bl1_ref[...], 0.0)
    out_ref[...] = (jnp.dot(hid, wl2_ref[...], preferred_element_type=jnp.float32, precision=lax.Precision.HIGHEST)
                    + bl2_ref[...])


_f32 = jnp.float32


def _pad_w(w):
    k, f = w.shape
    return jnp.pad(w, ((0, (FP - k) if k == 12 else 0), (0, FP - f)))


def _pad_v(v):
    return jnp.pad(v, (0, FP - v.shape[0])).reshape(1, FP)


def kernel(x, edge_index, W1_root, W1_rel, b1, g1, bt1,
           W2_root, W2_rel, b2, g2, bt2, Wl1, bl1, Wl2, bl2):
    src3 = edge_index[0].reshape(NW, NSTEP, CH)
    dst3 = edge_index[1].reshape(NW, NSTEP, CH)
    zeros_pad = jnp.zeros((NPAD, FP), _f32)

    w1rel = _pad_w(W1_rel)
    w1root = _pad_w(W1_root)
    w2rel = _pad_w(W2_rel)
    w2root = _pad_w(W2_root)

    m1, r1 = pl.pallas_call(
        _proj_body,
        out_shape=[jax.ShapeDtypeStruct((N, FP), _f32)] * 2,
    )(x, w1rel, w1root)

    segsum = _make_segsum_sc()
    part1 = segsum(m1, src3, dst3, zeros_pad)

    m2, r2 = pl.pallas_call(
        _mid_body,
        out_shape=[jax.ShapeDtypeStruct((N, FP), _f32)] * 2,
    )(r1, part1[0, :N], part1[1, :N], _pad_v(b1), _pad_v(g1), _pad_v(bt1),
      w2rel, w2root)

    part2 = segsum(m2, src3, dst3, zeros_pad)

    h2 = pl.pallas_call(
        _last_body,
        out_shape=jax.ShapeDtypeStruct((N, FP), _f32),
    )(r2, part2[0, :N], part2[1, :N], _pad_v(b2), _pad_v(g2), _pad_v(bt2))

    hf = h2[:, :12].reshape(BATCH, N_BUS * 12)

    out = pl.pallas_call(
        _head_body,
        out_shape=jax.ShapeDtypeStruct((BATCH, 2 * N_BUS), _f32),
    )(hf, Wl1, bl1.reshape(1, -1), Wl2, bl2.reshape(1, -1))
    return out


# speedup vs baseline: 13.7110x; 1.3582x over previous
import functools

import jax
import jax.numpy as jnp
from jax import lax
from jax.experimental import pallas as pl
from jax.experimental.pallas import tpu as pltpu
from jax.experimental.pallas import tpu_sc as plsc

N_BUS = 1000
BATCH = 10
N = N_BUS * BATCH
E = 320000
FP = 16
EPS = 1e-5

NC = 2
NS = 16
NW = NC * NS
CH = 128
NSTEP = 80
EPAD = NW * NSTEP * CH
DPAD = 10016
NB = 8
NGRP = NSTEP // NB
NPAD = 10240
RPW = NPAD // NS


@functools.cache
def _make_segsum_sc():
    mesh = plsc.VectorSubcoreMesh(core_axis_name="c", subcore_axis_name="s")
    return functools.partial(
        pl.kernel,
        mesh=mesh,
        compiler_params=pltpu.CompilerParams(use_tc_tiling_on_sc=False),
        out_type=jax.ShapeDtypeStruct((NC, NPAD, FP), jnp.float32),
        scratch_types=[
            pltpu.VMEM((NSTEP, CH), jnp.int32),
            pltpu.VMEM((NSTEP, CH), jnp.int32),
            [pltpu.VMEM((CH, FP), jnp.float32)] * NB,
            pltpu.VMEM_SHARED((NPAD, FP), jnp.float32),
            [pltpu.SemaphoreType.DMA] * NB,
            [pltpu.SemaphoreType.DMA] * NB,
        ],
    )(_segsum_body)


def _segsum_body(m_hbm, src_hbm, dst_hbm, zeros_hbm, out_hbm,
                 src_v, dst_v, rows, acc_sh, gsem, ssem):
    c = lax.axis_index("c")
    s = lax.axis_index("s")
    wid = c * NS + s

    pltpu.sync_copy(src_hbm.at[wid], src_v)
    pltpu.sync_copy(dst_hbm.at[wid], dst_v)

    for b in range(NB):
        pltpu.async_copy(m_hbm.at[src_v.at[b]], rows[b], gsem[b])
    pltpu.sync_copy(zeros_hbm.at[pl.ds(s * RPW, RPW)],
                    acc_sh.at[pl.ds(s * RPW, RPW)])
    plsc.subcore_barrier()

    def group(g, reissue):
        base = g * NB
        scatters = []
        for b in range(NB):
            j = base + b
            pltpu.make_async_copy(m_hbm.at[src_v.at[j]], rows[b],
                                  gsem[b]).wait()
            scatters.append(pltpu.async_copy(
                rows[b], acc_sh.at[dst_v.at[j]], ssem[b], add=True))
        for b in range(NB):
            scatters[b].wait()
            if reissue:
                pltpu.async_copy(m_hbm.at[src_v.at[base + NB + b]],
                                 rows[b], gsem[b])
        return 0

    lax.fori_loop(0, NGRP - 1, lambda g, _: group(g, True), 0)
    group(NGRP - 1, False)
    plsc.subcore_barrier()

    pltpu.sync_copy(acc_sh.at[pl.ds(s * RPW, RPW)],
                    out_hbm.at[c, pl.ds(s * RPW, RPW)])


def _proj_body(x_ref, wrel_ref, wroot_ref, m_ref, r_ref):
    x = x_ref[...]
    m_ref[...] = jnp.dot(x, wrel_ref[...], preferred_element_type=jnp.float32, precision=lax.Precision.HIGHEST)
    r_ref[...] = jnp.dot(x, wroot_ref[...], preferred_element_type=jnp.float32, precision=lax.Precision.HIGHEST)


def _bn_relu(h, g, bt):
    mu = jnp.mean(h, axis=0, keepdims=True)
    var = jnp.mean(h * h, axis=0, keepdims=True) - mu * mu
    hn = g * (h - mu) * lax.rsqrt(var + EPS) + bt
    return jnp.maximum(hn, 0.0)


def _mid_body(r_ref, p0_ref, p1_ref, b_ref, g_ref, bt_ref,
              wrel_ref, wroot_ref, m2_ref, r2_ref):
    h = r_ref[...] + p0_ref[...] + p1_ref[...] + b_ref[...]
    h1 = _bn_relu(h, g_ref[...], bt_ref[...])
    m2_ref[...] = jnp.dot(h1, wrel_ref[...], preferred_element_type=jnp.float32, precision=lax.Precision.HIGHEST)
    r2_ref[...] = jnp.dot(h1, wroot_ref[...], preferred_element_type=jnp.float32, precision=lax.Precision.HIGHEST)


def _last_body(r_ref, p0_ref, p1_ref, b_ref, g_ref, bt_ref, h2_ref):
    h = r_ref[...] + p0_ref[...] + p1_ref[...] + b_ref[...]
    h2_ref[...] = _bn_relu(h, g_ref[...], bt_ref[...])


def _head_body(hf_ref, wl1_ref, bl1_ref, wl2_ref, bl2_ref, out_ref):
    hid = jnp.dot(hf_ref[...], wl1_ref[...], preferred_element_type=jnp.float32, precision=lax.Precision.HIGHEST)
    hid = jnp.maximum(hid + bl1_ref[...], 0.0)
    out_ref[...] = (jnp.dot(hid, wl2_ref[...], preferred_element_type=jnp.float32, precision=lax.Precision.HIGHEST)
                    + bl2_ref[...])


_f32 = jnp.float32


def _pad_w(w):
    k, f = w.shape
    return jnp.pad(w, ((0, (FP - k) if k == 12 else 0), (0, FP - f)))


def _pad_v(v):
    return jnp.pad(v, (0, FP - v.shape[0])).reshape(1, FP)


def kernel(x, edge_index, W1_root, W1_rel, b1, g1, bt1,
           W2_root, W2_rel, b2, g2, bt2, Wl1, bl1, Wl2, bl2):
    pad_n = EPAD - E
    src3 = jnp.concatenate(
        [edge_index[0], jnp.zeros((pad_n,), jnp.int32)]).reshape(NW, NSTEP, CH)
    dst3 = jnp.concatenate(
        [edge_index[1], jnp.full((pad_n,), DPAD, jnp.int32)]).reshape(NW, NSTEP, CH)
    zeros_pad = jnp.zeros((NPAD, FP), _f32)

    w1rel = _pad_w(W1_rel)
    w1root = _pad_w(W1_root)
    w2rel = _pad_w(W2_rel)
    w2root = _pad_w(W2_root)

    m1, r1 = pl.pallas_call(
        _proj_body,
        out_shape=[jax.ShapeDtypeStruct((N, FP), _f32)] * 2,
    )(x, w1rel, w1root)

    segsum = _make_segsum_sc()
    part1 = segsum(m1, src3, dst3, zeros_pad)

    m2, r2 = pl.pallas_call(
        _mid_body,
        out_shape=[jax.ShapeDtypeStruct((N, FP), _f32)] * 2,
    )(r1, part1[0, :N], part1[1, :N], _pad_v(b1), _pad_v(g1), _pad_v(bt1),
      w2rel, w2root)

    part2 = segsum(m2, src3, dst3, zeros_pad)

    h2 = pl.pallas_call(
        _last_body,
        out_shape=jax.ShapeDtypeStruct((N, FP), _f32),
    )(r2, part2[0, :N], part2[1, :N], _pad_v(b2), _pad_v(g2), _pad_v(bt2))

    hf = h2[:, :12].reshape(BATCH, N_BUS * 12)

    out = pl.pallas_call(
        _head_body,
        out_shape=jax.ShapeDtypeStruct((BATCH, 2 * N_BUS), _f32),
    )(hf, Wl1, bl1.reshape(1, -1), Wl2, bl2.reshape(1, -1))
    return out
